# Initial kernel scaffold; baseline (speedup 1.0000x reference)
import jax
import jax.numpy as jnp
from jax.experimental import pallas as pl


def kernel(points, nsamples, kd_depth, return_gathered):
    raise NotImplementedError("write your pallas kernel here")


# speedup vs baseline: 28.7338x; 28.7338x over previous
import jax
import jax.numpy as jnp
from jax.experimental import pallas as pl
from jax.experimental.pallas import tpu as pltpu

_B, _P, _NS = 16, 16384, 1024


def _fps_body(pts_ref, idx_ref, sx_ref, sy_ref, sz_ref, dist_ref):
    iota = jax.lax.broadcasted_iota(jnp.int32, (_B, _P), 1)
    dist_ref[...] = jnp.full((_B, _P), 1e10, jnp.float32)

    px0 = pts_ref[0]
    nxt0 = jnp.zeros((_B, 1), jnp.int32)
    cx0 = px0[:, 0:1]
    cy0 = pts_ref[1, :, 0:1]
    cz0 = pts_ref[2, :, 0:1]

    def body(s, carry):
        nxt, cx, cy, cz = carry
        idx_ref[pl.ds(s, 1), :] = nxt.reshape(1, _B)
        sx_ref[pl.ds(s, 1), :] = cx.reshape(1, _B)
        sy_ref[pl.ds(s, 1), :] = cy.reshape(1, _B)
        sz_ref[pl.ds(s, 1), :] = cz.reshape(1, _B)

        px = pts_ref[0]
        py = pts_ref[1]
        pz = pts_ref[2]
        dx = px - cx
        dy = py - cy
        dz = pz - cz
        d = dx * dx + dy * dy + dz * dz
        dist = jnp.minimum(dist_ref[...], d)
        dist_ref[...] = dist

        m = jnp.max(dist, axis=1, keepdims=True)
        nxt2 = jnp.min(jnp.where(dist == m, iota, _P),
                       axis=1, keepdims=True).astype(jnp.int32)
        oh = iota == nxt2
        cx2 = jnp.sum(jnp.where(oh, px, 0.0), axis=1, keepdims=True)
        cy2 = jnp.sum(jnp.where(oh, py, 0.0), axis=1, keepdims=True)
        cz2 = jnp.sum(jnp.where(oh, pz, 0.0), axis=1, keepdims=True)
        return nxt2, cx2, cy2, cz2

    jax.lax.fori_loop(0, _NS, body, (nxt0, cx0, cy0, cz0))


def _run(points, interpret=False):
    pts = jnp.transpose(points, (2, 0, 1))
    idx_t, sx, sy, sz = pl.pallas_call(
        _fps_body,
        out_shape=[
            jax.ShapeDtypeStruct((_NS, _B), jnp.int32),
            jax.ShapeDtypeStruct((_NS, _B), jnp.float32),
            jax.ShapeDtypeStruct((_NS, _B), jnp.float32),
            jax.ShapeDtypeStruct((_NS, _B), jnp.float32),
        ],
        scratch_shapes=[pltpu.VMEM((_B, _P), jnp.float32)],
        interpret=interpret,
    )(pts)
    indices = jnp.transpose(idx_t)
    sampled = jnp.stack([jnp.transpose(sx), jnp.transpose(sy),
                         jnp.transpose(sz)], axis=-1)
    return indices, sampled


def kernel(points, nsamples, kd_depth, return_gathered):
    return _run(points)
